# baseline (speedup 1.0000x reference)
import dataclasses
import functools

import jax
import jax.numpy as jnp
from jax import lax
from jax.experimental import pallas as pl
from jax.experimental.pallas import tpu as pltpu
from jax.experimental.pallas import tpu_sc as plsc

NUM_ENTITIES = 1000000
NUM_RELATIONS = 1000
EMBED_DIM = 64
BATCH = 16384

NC = 2
NS = 16
NW = NC * NS
LANES = 16

B_PER_W = BATCH // NW
CHUNK = 128
N_CHUNKS = B_PER_W // CHUNK


def _trig_body(rr_ref, ri_ref, out_ref):
    rr = rr_ref[...]
    out_ref[...] = jnp.concatenate(
        [jnp.cos(rr), jnp.sin(rr), jnp.cos(ri_ref[...]),
         jnp.zeros_like(rr)], axis=1)


def _relation_trig(relation_re, relation_im):
    return pl.pallas_call(
        _trig_body,
        out_shape=jax.ShapeDtypeStruct((NUM_RELATIONS, 4 * EMBED_DIM),
                                       jnp.float32),
    )(relation_re, relation_im)


def _sc_score_kernel(h2_hbm, hp_hbm, t2_hbm, tp_hbm, r_hbm,
                     ere_hbm, eim_hbm, rel_hbm, out_hbm,
                     idx_h, idx_t, idx_r, par_h, par_t,
                     hr_v, hi_v, tr_v, ti_v, r_v, scores_v, sem):
    wid = lax.axis_index("s") * NC + lax.axis_index("c")
    base_w = wid * B_PER_W
    D = EMBED_DIM

    @pl.loop(0, N_CHUNKS)
    def _(chunk):
        base = base_w + chunk * CHUNK
        pltpu.sync_copy(h2_hbm.at[pl.ds(base, CHUNK)], idx_h)
        pltpu.sync_copy(t2_hbm.at[pl.ds(base, CHUNK)], idx_t)
        pltpu.sync_copy(r_hbm.at[pl.ds(base, CHUNK)], idx_r)
        pltpu.sync_copy(hp_hbm.at[pl.ds(base, CHUNK)], par_h)
        pltpu.sync_copy(tp_hbm.at[pl.ds(base, CHUNK)], par_t)
        copies = [
            pltpu.async_copy(ere_hbm.at[idx_h], hr_v, sem),
            pltpu.async_copy(eim_hbm.at[idx_h], hi_v, sem),
            pltpu.async_copy(ere_hbm.at[idx_t], tr_v, sem),
            pltpu.async_copy(eim_hbm.at[idx_t], ti_v, sem),
            pltpu.async_copy(rel_hbm.at[idx_r], r_v, sem),
        ]
        for cp_ in copies:
            cp_.wait()

        lane = lax.iota(jnp.int32, LANES)

        @pl.loop(0, CHUNK // LANES)
        def _(g):
            hp_vec = par_h[pl.ds(g * LANES, LANES)]
            tp_vec = par_t[pl.ds(g * LANES, LANES)]
            svec = jnp.zeros((LANES,), jnp.float32)
            for k in range(LANES):
                b = g * LANES + k
                bh = hp_vec[k] * D
                bt = tp_vec[k] * D
                acc = None
                for j in range(D // LANES):
                    o = j * LANES
                    trv = tr_v[b, pl.ds(bt + o, LANES)]
                    tiv = ti_v[b, pl.ds(bt + o, LANES)]
                    cc = r_v[b, pl.ds(o, LANES)]
                    ss = r_v[b, pl.ds(D + o, LANES)]
                    ci = r_v[b, pl.ds(2 * D + o, LANES)]
                    rot_r = trv * cc - tiv * ss
                    rot_i = trv * ss + tiv * ci
                    dr = hr_v[b, pl.ds(bh + o, LANES)] - rot_r
                    di = hi_v[b, pl.ds(bh + o, LANES)] - rot_i
                    part = dr * dr + di * di
                    acc = part if acc is None else acc + part
                svec = jnp.where(lane == k, jnp.sum(acc), svec)
            scores_v[pl.ds(g * LANES, LANES)] = svec

        pltpu.sync_copy(scores_v, out_hbm.at[pl.ds(base, CHUNK)])


def _sc_score(h2, hp, t2, tp, r, ere2, eim2, rel_tbl):
    mesh = plsc.VectorSubcoreMesh(core_axis_name="c", subcore_axis_name="s")
    cp = pltpu.CompilerParams()
    if "needs_layout_passes" in pltpu.CompilerParams.__dataclass_fields__:
        cp = dataclasses.replace(cp, needs_layout_passes=False)
    run = functools.partial(
        pl.kernel,
        mesh=mesh,
        compiler_params=cp,
        out_type=jax.ShapeDtypeStruct((BATCH,), jnp.float32),
        scratch_types=[
            pltpu.VMEM((CHUNK,), jnp.int32),
            pltpu.VMEM((CHUNK,), jnp.int32),
            pltpu.VMEM((CHUNK,), jnp.int32),
            pltpu.VMEM((CHUNK,), jnp.int32),
            pltpu.VMEM((CHUNK,), jnp.int32),
            pltpu.VMEM((CHUNK, 2 * EMBED_DIM), jnp.float32),
            pltpu.VMEM((CHUNK, 2 * EMBED_DIM), jnp.float32),
            pltpu.VMEM((CHUNK, 2 * EMBED_DIM), jnp.float32),
            pltpu.VMEM((CHUNK, 2 * EMBED_DIM), jnp.float32),
            pltpu.VMEM((CHUNK, 4 * EMBED_DIM), jnp.float32),
            pltpu.VMEM((CHUNK,), jnp.float32),
            pltpu.SemaphoreType.DMA,
        ],
    )(_sc_score_kernel)
    return run(h2, hp, t2, tp, r, ere2, eim2, rel_tbl)


def kernel(h, r, t, entity_re, entity_im, relation_re, relation_im):
    h = h.astype(jnp.int32)
    r = r.astype(jnp.int32)
    t = t.astype(jnp.int32)
    ere2 = jnp.reshape(entity_re, (NUM_ENTITIES // 2, 2 * EMBED_DIM))
    eim2 = jnp.reshape(entity_im, (NUM_ENTITIES // 2, 2 * EMBED_DIM))
    rel_tbl = _relation_trig(relation_re, relation_im)
    h2, hp = h >> 1, h & 1
    t2, tp = t >> 1, t & 1
    return _sc_score(h2, hp, t2, tp, r, ere2, eim2, rel_tbl)
